# baseline (speedup 1.0000x reference)
import functools

import jax
import jax.numpy as jnp
from jax import lax
from jax.experimental import pallas as pl
from jax.experimental.pallas import tpu as pltpu
from jax.experimental.pallas import tpu_sc as plsc

NC = 2
NS = 16
NW = NC * NS
LANES = 16
PADD = 128


def _dense_body(attr_ref, w_ref, b_ref, o_ref):
    o_ref[...] = lax.dot_general(
        attr_ref[...], w_ref[...],
        dimension_numbers=(((1,), (1,)), ((), ())),
        preferred_element_type=jnp.float32,
    ) + b_ref[...]


def _make_sc_kernel(Bsz, F, V, D):
    RPW = Bsz // NW
    R = 32
    NCH = RPW // R
    CL = R * F
    SEG = 4 * F
    NSEG = CL // SEG
    assert CL % SEG == 0 and CL % LANES == 0
    assert SEG <= 128 and SEG % 8 == 0
    HV = D // LANES

    mesh = plsc.VectorSubcoreMesh(
        core_axis_name="c", subcore_axis_name="s",
        num_cores=NC, num_subcores=NS,
    )

    @functools.partial(
        pl.kernel,
        out_type=jax.ShapeDtypeStruct((Bsz, PADD), jnp.float32),
        mesh=mesh,
        compiler_params=pltpu.CompilerParams(use_tc_tiling_on_sc=True),
        scratch_types=[
            pltpu.VMEM((CL + LANES,), jnp.int32),
            pltpu.VMEM((CL,), jnp.int32),
            pltpu.VMEM((CL,), jnp.int32),
            pltpu.VMEM((CL, PADD), jnp.float32),
            pltpu.VMEM((R, PADD), jnp.float32),
            pltpu.VMEM((R, PADD), jnp.float32),
            pltpu.SemaphoreType.DMA,
        ],
    )
    def sc_kernel(labels_hbm, offs_hbm, table_hbm, base_hbm, out_hbm,
                  lab_v, idx_v, offs_v, rows_v, base_v, out_v, sem):
        cid = lax.axis_index("c")
        sid = lax.axis_index("s")
        wid = sid * NC + cid
        row0 = wid * RPW

        pltpu.sync_copy(offs_hbm, offs_v)

        for g in range(NCH):
            r0 = row0 + g * R

            pltpu.sync_copy(labels_hbm.at[pl.ds(r0 * F, CL)],
                            lab_v.at[pl.ds(0, CL)])

            def idx_body(p, _):
                q = p * LANES
                idx_v[pl.ds(q, LANES)] = lax.shift_right_logical(
                    lab_v[pl.ds(q, LANES)] + offs_v[pl.ds(q, LANES)], 1
                )
                return 0
            lax.fori_loop(0, CL // LANES, idx_body, 0)

            pltpu.sync_copy(base_hbm.at[pl.ds(r0, R)], base_v)

            handles = [
                pltpu.async_copy(
                    table_hbm.at[idx_v.at[pl.ds(s * SEG, SEG)]],
                    rows_v.at[pl.ds(s * SEG, SEG)],
                    sem,
                )
                for s in range(NSEG)
            ]
            for h in handles:
                h.wait()

            def row_body(r, _):
                rb = r * F
                accs = [base_v[r, pl.ds(cc * LANES, LANES)]
                        for cc in range(HV)]
                for j in range(F):
                    lv = lab_v[pl.ds(rb + j, LANES)]
                    half = (lv[0] & 1) * D
                    for cc in range(HV):
                        accs[cc] = accs[cc] + rows_v[
                            rb + j, pl.ds(half + cc * LANES, LANES)]
                for cc in range(HV):
                    out_v[r, pl.ds(cc * LANES, LANES)] = accs[cc]
                return 0
            lax.fori_loop(0, R, row_body, 0)

            pltpu.sync_copy(out_v, out_hbm.at[pl.ds(r0, R)])

    return sc_kernel


@jax.jit
def kernel(labels, attr, tables, W, b):
    Bsz, F = labels.shape
    _, V, D = tables.shape

    Wp = jnp.zeros((PADD, W.shape[1]), jnp.float32).at[:D].set(W)
    bp = jnp.zeros((1, PADD), jnp.float32).at[0, :D].set(b)
    base = pl.pallas_call(
        _dense_body,
        out_shape=jax.ShapeDtypeStruct((Bsz, PADD), jnp.float32),
    )(attr, Wp, bp)

    labels_flat = labels.astype(jnp.int32).reshape(-1)
    table_pairs = tables.reshape(F * V // 2, PADD)
    offs = jnp.tile(jnp.arange(F, dtype=jnp.int32) * V, 32)

    sc = _make_sc_kernel(Bsz, F, V, D)
    out_full = sc(labels_flat, offs, table_pairs, base)
    return out_full[:, :D]


# speedup vs baseline: 1.5780x; 1.5780x over previous
import functools

import jax
import jax.numpy as jnp
from jax import lax
from jax.experimental import pallas as pl
from jax.experimental.pallas import tpu as pltpu
from jax.experimental.pallas import tpu_sc as plsc

NC = 2
NS = 16
NW = NC * NS
LANES = 16
PADD = 128


def _dense_body(attr_ref, w_ref, b_ref, o_ref):
    o_ref[...] = lax.dot_general(
        attr_ref[...], w_ref[...],
        dimension_numbers=(((1,), (1,)), ((), ())),
        preferred_element_type=jnp.float32,
    ) + b_ref[...]


DELTA = 51200
VCHUNK = 12800


def _pack_body(x_ref, eye_ref, o_hbm, y_ref, sem):
    f = pl.program_id(0)
    V = x_ref.shape[2]
    D = x_ref.shape[1]

    def _t(v0, w):
        xc = x_ref[0, :, pl.ds(v0, w)]
        return lax.dot_general(
            xc, eye_ref[...],
            dimension_numbers=(((0,), (0,)), ((), ())),
            preferred_element_type=jnp.float32,
        )

    for c in range(DELTA // VCHUNK):
        v0 = VCHUNK * c
        y_ref[:, pl.ds(0, D)] = _t(v0, VCHUNK)
        w1 = min(VCHUNK, V - DELTA - v0)
        y_ref[pl.ds(0, w1), pl.ds(D, D)] = _t(DELTA + v0, w1)
        cp = pltpu.make_async_copy(
            y_ref,
            o_hbm.at[pl.ds(f * DELTA + v0, VCHUNK)],
            sem,
        )
        cp.start()
        cp.wait()


def _make_sc_kernel(Bsz, F, V, D):
    RPW = Bsz // NW
    R = 32
    NCH = RPW // R
    CL = R * F
    SEG = 4 * F
    NSEG = CL // SEG
    assert CL % SEG == 0 and CL % LANES == 0
    assert SEG <= 128 and SEG % 8 == 0
    HV = D // LANES

    mesh = plsc.VectorSubcoreMesh(
        core_axis_name="c", subcore_axis_name="s",
        num_cores=NC, num_subcores=NS,
    )

    @functools.partial(
        pl.kernel,
        out_type=jax.ShapeDtypeStruct((Bsz, PADD), jnp.float32),
        mesh=mesh,
        compiler_params=pltpu.CompilerParams(use_tc_tiling_on_sc=True),
        scratch_types=[
            pltpu.VMEM((CL + LANES,), jnp.int32),
            pltpu.VMEM((CL,), jnp.int32),
            pltpu.VMEM((CL,), jnp.int32),
            pltpu.VMEM((CL, PADD), jnp.float32),
            pltpu.VMEM((R, PADD), jnp.float32),
            pltpu.VMEM((R, PADD), jnp.float32),
            pltpu.SemaphoreType.DMA,
        ],
    )
    def sc_kernel(labels_hbm, offs_hbm, table_hbm, base_hbm, out_hbm,
                  lab_v, idx_v, offs_v, rows_v, base_v, out_v, sem):
        cid = lax.axis_index("c")
        sid = lax.axis_index("s")
        wid = sid * NC + cid
        row0 = wid * RPW

        pltpu.sync_copy(offs_hbm, offs_v)

        for g in range(NCH):
            r0 = row0 + g * R

            pltpu.sync_copy(labels_hbm.at[pl.ds(r0 * F, CL)],
                            lab_v.at[pl.ds(0, CL)])

            def idx_body(p, _):
                q = p * LANES
                lv = lab_v[pl.ds(q, LANES)]
                vmod = lv - jnp.where(lv >= DELTA, DELTA, 0)
                idx_v[pl.ds(q, LANES)] = vmod + offs_v[pl.ds(q, LANES)]
                return 0
            lax.fori_loop(0, CL // LANES, idx_body, 0)

            pltpu.sync_copy(base_hbm.at[pl.ds(r0, R)], base_v)

            handles = [
                pltpu.async_copy(
                    table_hbm.at[idx_v.at[pl.ds(s * SEG, SEG)]],
                    rows_v.at[pl.ds(s * SEG, SEG)],
                    sem,
                )
                for s in range(NSEG)
            ]
            for h in handles:
                h.wait()

            def row_body(r, _):
                rb = r * F
                accs = [base_v[r, pl.ds(cc * LANES, LANES)]
                        for cc in range(HV)]
                for j in range(F):
                    lv = lab_v[pl.ds(rb + j, LANES)]
                    half = jnp.where(lv[0] >= DELTA, D, 0)
                    for cc in range(HV):
                        accs[cc] = accs[cc] + rows_v[
                            rb + j, pl.ds(half + cc * LANES, LANES)]
                for cc in range(HV):
                    out_v[r, pl.ds(cc * LANES, LANES)] = accs[cc]
                return 0
            lax.fori_loop(0, R, row_body, 0)

            pltpu.sync_copy(out_v, out_hbm.at[pl.ds(r0, R)])

    return sc_kernel


@jax.jit
def kernel(labels, attr, tables, W, b):
    Bsz, F = labels.shape
    _, V, D = tables.shape
    NH = D // LANES

    Wp = jnp.zeros((PADD, W.shape[1]), jnp.float32).at[:D].set(W)
    bp = jnp.zeros((1, PADD), jnp.float32).at[0, :D].set(b)
    base = pl.pallas_call(
        _dense_body,
        out_shape=jax.ShapeDtypeStruct((Bsz, PADD), jnp.float32),
    )(attr, Wp, bp)

    tables_T = jnp.transpose(tables, (0, 2, 1))
    eye = jnp.eye(D, dtype=jnp.float32)
    pairs = pl.pallas_call(
        _pack_body,
        grid=(F,),
        in_specs=[
            pl.BlockSpec((1, D, V), lambda f: (f, 0, 0)),
            pl.BlockSpec((D, D), lambda f: (0, 0)),
        ],
        out_specs=pl.BlockSpec(memory_space=pl.ANY),
        out_shape=jax.ShapeDtypeStruct((F * DELTA, PADD), jnp.float32),
        scratch_shapes=[
            pltpu.VMEM((VCHUNK, PADD), jnp.float32),
            pltpu.SemaphoreType.DMA,
        ],
        compiler_params=pltpu.CompilerParams(
            vmem_limit_bytes=62 * 1024 * 1024),
    )(tables_T, eye)

    labels_flat = labels.astype(jnp.int32).reshape(-1)
    offs = jnp.tile(jnp.arange(F, dtype=jnp.int32) * DELTA, 32)

    sc = _make_sc_kernel(Bsz, F, V, D)
    out_full = sc(labels_flat, offs, pairs, base)
    return out_full[:, :D]


# speedup vs baseline: 2.5169x; 1.5950x over previous
import functools

import jax
import jax.numpy as jnp
from jax import lax
from jax.experimental import pallas as pl
from jax.experimental.pallas import tpu as pltpu
from jax.experimental.pallas import tpu_sc as plsc

NC = 2
NS = 16
NW = NC * NS
LANES = 16
PADD = 128


def _dense_body(attr_ref, w_ref, b_ref, o_ref):
    o_ref[...] = lax.dot_general(
        attr_ref[...], w_ref[...],
        dimension_numbers=(((1,), (1,)), ((), ())),
        preferred_element_type=jnp.float32,
    ) + b_ref[...]


DELTA = 51200
VCHUNK = 6400


def _pack_body(x_ref, eye_ref, o_hbm, y_ref, sem):
    f = pl.program_id(0)
    V = x_ref.shape[2]
    D = x_ref.shape[1]

    def _dot(x2, eye):
        return lax.dot_general(
            x2, eye,
            dimension_numbers=(((0,), (0,)), ((), ())),
            preferred_element_type=jnp.float32,
        )

    cps = []
    for c in range(DELTA // VCHUNK):
        v0 = VCHUNK * c
        w1 = min(VCHUNK, V - DELTA - v0)
        yb = y_ref.at[c % 2]
        if len(cps) >= 2:
            cps[-2].wait()
        lo = x_ref[0, :, pl.ds(v0, VCHUNK)]
        if w1 == VCHUNK:
            hi = x_ref[0, :, pl.ds(DELTA + v0, VCHUNK)]
            x2 = jnp.concatenate([lo, hi], axis=0)
            yb[...] = _dot(x2, eye_ref[...])
        else:
            yb[:, pl.ds(0, D)] = _dot(lo, eye_ref[pl.ds(0, D),
                                                  pl.ds(0, D)])
            hi = x_ref[0, :, pl.ds(DELTA + v0, w1)]
            yb[pl.ds(0, w1), pl.ds(D, D)] = _dot(
                hi, eye_ref[pl.ds(0, D), pl.ds(0, D)])
        cp = pltpu.make_async_copy(
            yb,
            o_hbm.at[pl.ds(f * DELTA + v0, VCHUNK)],
            sem,
        )
        cp.start()
        cps.append(cp)
    for cp in cps[-2:]:
        cp.wait()


def _make_sc_kernel(Bsz, F, V, D):
    RPW = Bsz // NW
    R = 32
    NCH = RPW // R
    CL = R * F
    SEG = 4 * F
    NSEG = CL // SEG
    assert CL % SEG == 0 and CL % LANES == 0
    assert SEG <= 128 and SEG % 8 == 0
    HV = D // LANES

    mesh = plsc.VectorSubcoreMesh(
        core_axis_name="c", subcore_axis_name="s",
        num_cores=NC, num_subcores=NS,
    )

    @functools.partial(
        pl.kernel,
        out_type=jax.ShapeDtypeStruct((Bsz, PADD), jnp.float32),
        mesh=mesh,
        compiler_params=pltpu.CompilerParams(use_tc_tiling_on_sc=True),
        scratch_types=[
            pltpu.VMEM((CL + LANES,), jnp.int32),
            pltpu.VMEM((CL,), jnp.int32),
            pltpu.VMEM((CL,), jnp.int32),
            pltpu.VMEM((CL, PADD), jnp.float32),
            pltpu.VMEM((R, PADD), jnp.float32),
            pltpu.VMEM((R, PADD), jnp.float32),
            pltpu.SemaphoreType.DMA,
        ],
    )
    def sc_kernel(labels_hbm, offs_hbm, table_hbm, base_hbm, out_hbm,
                  lab_v, idx_v, offs_v, rows_v, base_v, out_v, sem):
        cid = lax.axis_index("c")
        sid = lax.axis_index("s")
        wid = sid * NC + cid
        row0 = wid * RPW

        pltpu.sync_copy(offs_hbm, offs_v)

        for g in range(NCH):
            r0 = row0 + g * R

            pltpu.sync_copy(labels_hbm.at[pl.ds(r0 * F, CL)],
                            lab_v.at[pl.ds(0, CL)])

            def idx_body(p, _):
                q = p * LANES
                lv = lab_v[pl.ds(q, LANES)]
                vmod = lv - jnp.where(lv >= DELTA, DELTA, 0)
                idx_v[pl.ds(q, LANES)] = vmod + offs_v[pl.ds(q, LANES)]
                return 0
            lax.fori_loop(0, CL // LANES, idx_body, 0)

            pltpu.sync_copy(base_hbm.at[pl.ds(r0, R)], base_v)

            handles = [
                pltpu.async_copy(
                    table_hbm.at[idx_v.at[pl.ds(s * SEG, SEG)]],
                    rows_v.at[pl.ds(s * SEG, SEG)],
                    sem,
                )
                for s in range(NSEG)
            ]
            for h in handles:
                h.wait()

            def row_body(r, _):
                rb = r * F
                accs = [base_v[r, pl.ds(cc * LANES, LANES)]
                        for cc in range(HV)]
                for j in range(F):
                    lv = lab_v[pl.ds(rb + j, LANES)]
                    half = jnp.where(lv[0] >= DELTA, D, 0)
                    for cc in range(HV):
                        accs[cc] = accs[cc] + rows_v[
                            rb + j, pl.ds(half + cc * LANES, LANES)]
                for cc in range(HV):
                    out_v[r, pl.ds(cc * LANES, LANES)] = accs[cc]
                return 0
            lax.fori_loop(0, R, row_body, 0)

            pltpu.sync_copy(out_v, out_hbm.at[pl.ds(r0, R)])

    return sc_kernel


@jax.jit
def kernel(labels, attr, tables, W, b):
    Bsz, F = labels.shape
    _, V, D = tables.shape
    NH = D // LANES

    Wp = jnp.zeros((PADD, W.shape[1]), jnp.float32).at[:D].set(W)
    bp = jnp.zeros((1, PADD), jnp.float32).at[0, :D].set(b)
    base = pl.pallas_call(
        _dense_body,
        out_shape=jax.ShapeDtypeStruct((Bsz, PADD), jnp.float32),
    )(attr, Wp, bp)

    tables_T = jnp.transpose(tables, (0, 2, 1))
    eye = jnp.eye(PADD, dtype=jnp.float32)
    pairs = pl.pallas_call(
        _pack_body,
        grid=(F,),
        in_specs=[
            pl.BlockSpec((1, D, V), lambda f: (f, 0, 0)),
            pl.BlockSpec((PADD, PADD), lambda f: (0, 0)),
        ],
        out_specs=pl.BlockSpec(memory_space=pl.ANY),
        out_shape=jax.ShapeDtypeStruct((F * DELTA, PADD), jnp.float32),
        scratch_shapes=[
            pltpu.VMEM((2, VCHUNK, PADD), jnp.float32),
            pltpu.SemaphoreType.DMA,
        ],
        compiler_params=pltpu.CompilerParams(
            vmem_limit_bytes=62 * 1024 * 1024),
    )(tables_T, eye)

    labels_flat = labels.astype(jnp.int32).reshape(-1)
    offs = jnp.tile(jnp.arange(F, dtype=jnp.int32) * DELTA, 32)

    sc = _make_sc_kernel(Bsz, F, V, D)
    out_full = sc(labels_flat, offs, pairs, base)
    return out_full[:, :D]


# speedup vs baseline: 2.6878x; 1.0679x over previous
import functools

import jax
import jax.numpy as jnp
from jax import lax
from jax.experimental import pallas as pl
from jax.experimental.pallas import tpu as pltpu
from jax.experimental.pallas import tpu_sc as plsc

NC = 2
NS = 16
NW = NC * NS
LANES = 16
PADD = 128


def _dense_body(attr_ref, w_ref, b_ref, o_ref):
    o_ref[...] = lax.dot_general(
        attr_ref[...], w_ref[...],
        dimension_numbers=(((1,), (1,)), ((), ())),
        preferred_element_type=jnp.float32,
    ) + b_ref[...]


DELTA = 51200
VCHUNK = 6400


def _pack_body(x_ref, eye_ref, o_hbm, y_ref, sem):
    f = pl.program_id(0)
    V = x_ref.shape[2]
    D = x_ref.shape[1]

    def _dot(x2, eye):
        return lax.dot_general(
            x2, eye,
            dimension_numbers=(((0,), (0,)), ((), ())),
            preferred_element_type=jnp.float32,
        )

    cps = []
    for c in range(DELTA // VCHUNK):
        v0 = VCHUNK * c
        w1 = min(VCHUNK, V - DELTA - v0)
        yb = y_ref.at[c % 2]
        if len(cps) >= 2:
            cps[-2].wait()
        lo = x_ref[0, :, pl.ds(v0, VCHUNK)]
        if w1 == VCHUNK:
            hi = x_ref[0, :, pl.ds(DELTA + v0, VCHUNK)]
            x2 = jnp.concatenate([lo, hi], axis=0)
            yb[...] = _dot(x2, eye_ref[...])
        else:
            yb[:, pl.ds(0, D)] = _dot(lo, eye_ref[pl.ds(0, D),
                                                  pl.ds(0, D)])
            hi = x_ref[0, :, pl.ds(DELTA + v0, w1)]
            yb[pl.ds(0, w1), pl.ds(D, D)] = _dot(
                hi, eye_ref[pl.ds(0, D), pl.ds(0, D)])
        cp = pltpu.make_async_copy(
            yb,
            o_hbm.at[pl.ds(f * DELTA + v0, VCHUNK)],
            sem,
        )
        cp.start()
        cps.append(cp)
    for cp in cps[-2:]:
        cp.wait()


def _make_sc_kernel(Bsz, F, V, D):
    RPW = Bsz // NW
    R = 32
    NCH = RPW // R
    CL = R * F
    SEG = 4 * F
    NSEG = CL // SEG
    assert CL % SEG == 0 and CL % LANES == 0
    assert SEG <= 128 and SEG % 8 == 0
    HV = D // LANES

    mesh = plsc.VectorSubcoreMesh(
        core_axis_name="c", subcore_axis_name="s",
        num_cores=NC, num_subcores=NS,
    )

    @functools.partial(
        pl.kernel,
        out_type=jax.ShapeDtypeStruct((Bsz, PADD), jnp.float32),
        mesh=mesh,
        compiler_params=pltpu.CompilerParams(use_tc_tiling_on_sc=True),
        scratch_types=[
            pltpu.VMEM((CL + LANES,), jnp.int32),
            pltpu.VMEM((CL,), jnp.int32),
            pltpu.VMEM((CL,), jnp.int32),
            pltpu.VMEM((CL, PADD), jnp.float32),
            pltpu.VMEM((R, PADD), jnp.float32),
            pltpu.VMEM((R, PADD), jnp.float32),
            pltpu.SemaphoreType.DMA,
        ],
    )
    def sc_kernel(labels_hbm, offs_hbm, table_hbm, base_hbm, out_hbm,
                  lab_v, idx_v, offs_v, rows_v, base_v, out_v, sem):
        cid = lax.axis_index("c")
        sid = lax.axis_index("s")
        wid = sid * NC + cid
        row0 = wid * RPW

        pltpu.sync_copy(offs_hbm, offs_v)

        def chunk_body(g, _):
            r0 = row0 + g * R

            pltpu.sync_copy(labels_hbm.at[pl.ds(r0 * F, CL)],
                            lab_v.at[pl.ds(0, CL)])

            def idx_body(p, _):
                q = p * LANES
                lv = lab_v[pl.ds(q, LANES)]
                vmod = lv - jnp.where(lv >= DELTA, DELTA, 0)
                idx_v[pl.ds(q, LANES)] = vmod + offs_v[pl.ds(q, LANES)]
                return 0
            lax.fori_loop(0, CL // LANES, idx_body, 0)

            pltpu.sync_copy(base_hbm.at[pl.ds(r0, R)], base_v)

            handles = [
                pltpu.async_copy(
                    table_hbm.at[idx_v.at[pl.ds(s * SEG, SEG)]],
                    rows_v.at[pl.ds(s * SEG, SEG)],
                    sem,
                )
                for s in range(NSEG)
            ]

            RSEG = SEG // F

            def seg_body(s, _):
                pltpu.make_async_copy(
                    table_hbm.at[idx_v.at[pl.ds(s * SEG, SEG)]],
                    rows_v.at[pl.ds(s * SEG, SEG)],
                    sem,
                ).wait()

                def row_body(r, _):
                    rb = r * F
                    accs = [base_v[r, pl.ds(cc * LANES, LANES)]
                            for cc in range(HV)]
                    for j in range(F):
                        lv = lab_v[pl.ds(rb + j, LANES)]
                        half = jnp.where(lv[0] >= DELTA, D, 0)
                        for cc in range(HV):
                            accs[cc] = accs[cc] + rows_v[
                                rb + j, pl.ds(half + cc * LANES, LANES)]
                    for cc in range(HV):
                        out_v[r, pl.ds(cc * LANES, LANES)] = accs[cc]
                    return 0
                lax.fori_loop(s * RSEG, (s + 1) * RSEG, row_body, 0)
                return 0
            lax.fori_loop(0, NSEG, seg_body, 0)
            del handles

            pltpu.sync_copy(out_v, out_hbm.at[pl.ds(r0, R)])
            return 0

        lax.fori_loop(0, NCH, chunk_body, 0)

    return sc_kernel


@jax.jit
def kernel(labels, attr, tables, W, b):
    Bsz, F = labels.shape
    _, V, D = tables.shape
    NH = D // LANES

    Wp = jnp.zeros((PADD, W.shape[1]), jnp.float32).at[:D].set(W)
    bp = jnp.zeros((1, PADD), jnp.float32).at[0, :D].set(b)
    base = pl.pallas_call(
        _dense_body,
        out_shape=jax.ShapeDtypeStruct((Bsz, PADD), jnp.float32),
    )(attr, Wp, bp)

    tables_T = jnp.transpose(tables, (0, 2, 1))
    eye = jnp.eye(PADD, dtype=jnp.float32)
    pairs = pl.pallas_call(
        _pack_body,
        grid=(F,),
        in_specs=[
            pl.BlockSpec((1, D, V), lambda f: (f, 0, 0)),
            pl.BlockSpec((PADD, PADD), lambda f: (0, 0)),
        ],
        out_specs=pl.BlockSpec(memory_space=pl.ANY),
        out_shape=jax.ShapeDtypeStruct((F * DELTA, PADD), jnp.float32),
        scratch_shapes=[
            pltpu.VMEM((2, VCHUNK, PADD), jnp.float32),
            pltpu.SemaphoreType.DMA,
        ],
        compiler_params=pltpu.CompilerParams(
            vmem_limit_bytes=62 * 1024 * 1024),
    )(tables_T, eye)

    labels_flat = labels.astype(jnp.int32).reshape(-1)
    offs = jnp.tile(jnp.arange(F, dtype=jnp.int32) * DELTA, 32)

    sc = _make_sc_kernel(Bsz, F, V, D)
    out_full = sc(labels_flat, offs, pairs, base)
    return out_full[:, :D]


# speedup vs baseline: 2.8172x; 1.0481x over previous
import functools

import jax
import jax.numpy as jnp
from jax import lax
from jax.experimental import pallas as pl
from jax.experimental.pallas import tpu as pltpu
from jax.experimental.pallas import tpu_sc as plsc

NC = 2
NS = 16
NW = NC * NS
LANES = 16
PADD = 128


def _dense_body(attr_ref, w_ref, b_ref, o_ref):
    o_ref[...] = lax.dot_general(
        attr_ref[...], w_ref[...],
        dimension_numbers=(((1,), (1,)), ((), ())),
        preferred_element_type=jnp.float32,
    ) + b_ref[...]


DELTA = 51200
VCHUNK = 6400


def _pack_body(x_ref, eye_ref, o_hbm, y_ref, sem):
    f = pl.program_id(0)
    V = x_ref.shape[2]
    D = x_ref.shape[1]

    def _dot(x2, eye):
        return lax.dot_general(
            x2, eye,
            dimension_numbers=(((0,), (0,)), ((), ())),
            preferred_element_type=jnp.float32,
        )

    cps = []
    for c in range(DELTA // VCHUNK):
        v0 = VCHUNK * c
        w1 = min(VCHUNK, V - DELTA - v0)
        yb = y_ref.at[c % 3]
        if len(cps) >= 3:
            cps[-3].wait()
        lo = x_ref[0, :, pl.ds(v0, VCHUNK)]
        if w1 == VCHUNK:
            hi = x_ref[0, :, pl.ds(DELTA + v0, VCHUNK)]
            x2 = jnp.concatenate([lo, hi], axis=0)
            yb[...] = _dot(x2, eye_ref[...])
        else:
            yb[:, pl.ds(0, D)] = _dot(lo, eye_ref[pl.ds(0, D),
                                                  pl.ds(0, D)])
            hi = x_ref[0, :, pl.ds(DELTA + v0, w1)]
            yb[pl.ds(0, w1), pl.ds(D, D)] = _dot(
                hi, eye_ref[pl.ds(0, D), pl.ds(0, D)])
        cp = pltpu.make_async_copy(
            yb,
            o_hbm.at[pl.ds(f * DELTA + v0, VCHUNK)],
            sem,
        )
        cp.start()
        cps.append(cp)
    for cp in cps[-3:]:
        cp.wait()


def _make_sc_kernel(Bsz, F, V, D):
    RPW = Bsz // NW
    R = 32
    NCH = RPW // R
    CL = R * F
    SEG = 4 * F
    NSEG = CL // SEG
    assert CL % SEG == 0 and CL % LANES == 0
    assert SEG <= 128 and SEG % 8 == 0
    HV = D // LANES

    mesh = plsc.VectorSubcoreMesh(
        core_axis_name="c", subcore_axis_name="s",
        num_cores=NC, num_subcores=NS,
    )

    @functools.partial(
        pl.kernel,
        out_type=jax.ShapeDtypeStruct((Bsz, PADD), jnp.float32),
        mesh=mesh,
        compiler_params=pltpu.CompilerParams(use_tc_tiling_on_sc=True),
        scratch_types=[
            pltpu.VMEM((CL + LANES,), jnp.int32),
            pltpu.VMEM((CL,), jnp.int32),
            pltpu.VMEM((CL,), jnp.int32),
            pltpu.VMEM((CL, PADD), jnp.float32),
            pltpu.VMEM((R, PADD), jnp.float32),
            pltpu.VMEM((R, PADD), jnp.float32),
            pltpu.SemaphoreType.DMA,
        ],
    )
    def sc_kernel(labels_hbm, offs_hbm, table_hbm, base_hbm, out_hbm,
                  lab_v, idx_v, offs_v, rows_v, base_v, out_v, sem):
        cid = lax.axis_index("c")
        sid = lax.axis_index("s")
        wid = sid * NC + cid
        row0 = wid * RPW

        pltpu.sync_copy(offs_hbm, offs_v)

        def chunk_body(g, _):
            r0 = row0 + g * R

            pltpu.sync_copy(labels_hbm.at[pl.ds(r0 * F, CL)],
                            lab_v.at[pl.ds(0, CL)])

            def idx_body(p, _):
                q = p * LANES
                lv = lab_v[pl.ds(q, LANES)]
                vmod = lv - jnp.where(lv >= DELTA, DELTA, 0)
                idx_v[pl.ds(q, LANES)] = vmod + offs_v[pl.ds(q, LANES)]
                return 0
            lax.fori_loop(0, CL // LANES, idx_body, 0)

            pltpu.sync_copy(base_hbm.at[pl.ds(r0, R)], base_v)

            handles = [
                pltpu.async_copy(
                    table_hbm.at[idx_v.at[pl.ds(s * SEG, SEG)]],
                    rows_v.at[pl.ds(s * SEG, SEG)],
                    sem,
                )
                for s in range(NSEG)
            ]

            RSEG = SEG // F

            def seg_body(s, _):
                pltpu.make_async_copy(
                    table_hbm.at[idx_v.at[pl.ds(s * SEG, SEG)]],
                    rows_v.at[pl.ds(s * SEG, SEG)],
                    sem,
                ).wait()

                def row_body(r, _):
                    rb = r * F
                    accs = [base_v[r, pl.ds(cc * LANES, LANES)]
                            for cc in range(HV)]
                    for j in range(F):
                        lv = lab_v[pl.ds(rb + j, LANES)]
                        half = jnp.where(lv[0] >= DELTA, D, 0)
                        for cc in range(HV):
                            accs[cc] = accs[cc] + rows_v[
                                rb + j, pl.ds(half + cc * LANES, LANES)]
                    for cc in range(HV):
                        out_v[r, pl.ds(cc * LANES, LANES)] = accs[cc]
                    return 0
                lax.fori_loop(s * RSEG, (s + 1) * RSEG, row_body, 0)
                return 0
            lax.fori_loop(0, NSEG, seg_body, 0)
            del handles

            pltpu.sync_copy(out_v, out_hbm.at[pl.ds(r0, R)])
            return 0

        lax.fori_loop(0, NCH, chunk_body, 0)

    return sc_kernel


@jax.jit
def kernel(labels, attr, tables, W, b):
    Bsz, F = labels.shape
    _, V, D = tables.shape
    NH = D // LANES

    Wp = jnp.zeros((PADD, W.shape[1]), jnp.float32).at[:D].set(W)
    bp = jnp.zeros((1, PADD), jnp.float32).at[0, :D].set(b)
    base = pl.pallas_call(
        _dense_body,
        out_shape=jax.ShapeDtypeStruct((Bsz, PADD), jnp.float32),
    )(attr, Wp, bp)

    tables_T = jnp.transpose(tables, (0, 2, 1))
    eye = jnp.eye(PADD, dtype=jnp.float32)
    pairs = pl.pallas_call(
        _pack_body,
        grid=(F,),
        in_specs=[
            pl.BlockSpec((1, D, V), lambda f: (f, 0, 0)),
            pl.BlockSpec((PADD, PADD), lambda f: (0, 0)),
        ],
        out_specs=pl.BlockSpec(memory_space=pl.ANY),
        out_shape=jax.ShapeDtypeStruct((F * DELTA, PADD), jnp.float32),
        scratch_shapes=[
            pltpu.VMEM((3, VCHUNK, PADD), jnp.float32),
            pltpu.SemaphoreType.DMA,
        ],
        compiler_params=pltpu.CompilerParams(
            vmem_limit_bytes=62 * 1024 * 1024),
    )(tables_T, eye)

    labels_flat = labels.astype(jnp.int32).reshape(-1)
    offs = jnp.tile(jnp.arange(F, dtype=jnp.int32) * DELTA, 32)

    sc = _make_sc_kernel(Bsz, F, V, D)
    out_full = sc(labels_flat, offs, pairs, base)
    return out_full[:, :D]
